# baseline (speedup 1.0000x reference)
import functools

import jax
import jax.numpy as jnp
from jax import lax
from jax.experimental import pallas as pl
from jax.experimental.pallas import tpu as pltpu
from jax.experimental.pallas import tpu_sc as plsc

E = 8
D = 1024
DFF = 4096
T = 4096
C = 1024
EC = E * C
TRASH = EC
TB = 512
NB = T // TB
KB = 2
DFFB = DFF // KB


def _routing_body(x_ref, wg_ref,
                  dst1_ref, dst2_ref, slot1_ref, slot2_ref, w1_ref, w2_ref,
                  fill_ref, aux_ref, cnt_ref,
                  me_s, c1_s, c2_s, r1_s, r2_s):
    p = pl.program_id(0)
    i = pl.program_id(1)
    x = x_ref[...]
    wg = wg_ref[...]
    logits = jnp.dot(x, wg, preferred_element_type=jnp.float32)
    mx = jnp.max(logits, axis=1, keepdims=True)
    ex = jnp.exp(logits - mx)
    gates = ex / jnp.sum(ex, axis=1, keepdims=True)
    eio = lax.broadcasted_iota(jnp.int32, (TB, E), 1)
    is1 = logits == mx
    idx1 = jnp.min(jnp.where(is1, eio, E), axis=1)
    mask1 = (eio == idx1[:, None]).astype(jnp.float32)
    lneg = jnp.where(mask1 > 0, -1e30, logits)
    mx2 = jnp.max(lneg, axis=1, keepdims=True)
    is2 = lneg == mx2
    idx2 = jnp.min(jnp.where(is2, eio, E), axis=1)
    mask2 = (eio == idx2[:, None]).astype(jnp.float32)

    @pl.when((p == 0) & (i == 0))
    def _():
        me_s[...] = jnp.zeros_like(me_s)
        c1_s[...] = jnp.zeros_like(c1_s)
        c2_s[...] = jnp.zeros_like(c2_s)

    @pl.when(p == 0)
    def _():
        me_s[...] += jnp.sum(gates, axis=0, keepdims=True)
        c1_s[...] += jnp.sum(mask1, axis=0, keepdims=True)
        c2_s[...] += jnp.sum(mask2, axis=0, keepdims=True)

    @pl.when((p == 1) & (i == 0))
    def _():
        r1_s[...] = jnp.zeros_like(r1_s)
        r2_s[...] = c1_s[...]
        aux_ref[...] = jnp.full(
            (1, 1), 8.0 * jnp.sum(me_s[...] * c1_s[...]) / float(T * T),
            jnp.float32)
        cnt_ref[...] = c1_s[...].astype(jnp.int32)
        fill_ref[...] = jnp.minimum(c1_s[...] + c2_s[...],
                                    float(C)).astype(jnp.int32)

    @pl.when(p == 1)
    def _():
        rio = lax.broadcasted_iota(jnp.int32, (TB, TB), 0)
        cio = lax.broadcasted_iota(jnp.int32, (TB, TB), 1)
        ltri = (rio >= cio).astype(jnp.float32)
        inc1 = jnp.dot(ltri, mask1, precision=lax.Precision.HIGHEST,
                       preferred_element_type=jnp.float32)
        inc2 = jnp.dot(ltri, mask2, precision=lax.Precision.HIGHEST,
                       preferred_element_type=jnp.float32)
        loc1 = inc1 - mask1 + r1_s[...]
        loc2 = inc2 - mask2 + r2_s[...]
        r1_s[...] += jnp.sum(mask1, axis=0, keepdims=True)
        r2_s[...] += jnp.sum(mask2, axis=0, keepdims=True)
        m1d = mask1 * (loc1 < C).astype(jnp.float32)
        m2d = mask2 * (loc2 < C).astype(jnp.float32)
        loc1s = jnp.sum(loc1 * m1d, axis=1)
        loc2s = jnp.sum(loc2 * m2d, axis=1)
        keep1 = jnp.sum(m1d, axis=1)
        keep2 = jnp.sum(m2d, axis=1)
        g1 = jnp.sum(gates * m1d, axis=1)
        g2 = jnp.sum(gates * m2d, axis=1)
        den = jnp.maximum(g1 + g2, jnp.finfo(jnp.float32).eps)
        wc1 = g1 / den * keep1
        wc2 = g2 / den * keep2
        loc1c = jnp.clip(loc1s.astype(jnp.int32), 0, C - 1)
        loc2c = jnp.clip(loc2s.astype(jnp.int32), 0, C - 1)
        slot1 = idx1 * C + loc1c
        slot2 = idx2 * C + loc2c
        sl = pl.ds(i * TB, TB)
        slot1_ref[0, sl] = slot1
        slot2_ref[0, sl] = slot2
        dst1_ref[0, sl] = jnp.where(keep1 > 0, slot1, TRASH)
        dst2_ref[0, sl] = jnp.where(keep2 > 0, slot2, TRASH)
        rsl = pl.ds(i * TB, TB)
        w1_ref[rsl, :] = jnp.broadcast_to(wc1[:, None], (TB, 16))
        w2_ref[rsl, :] = jnp.broadcast_to(wc2[:, None], (TB, 16))


def _routing(xr, wg):
    out_shapes = (
        jax.ShapeDtypeStruct((1, T), jnp.int32),
        jax.ShapeDtypeStruct((1, T), jnp.int32),
        jax.ShapeDtypeStruct((1, T), jnp.int32),
        jax.ShapeDtypeStruct((1, T), jnp.int32),
        jax.ShapeDtypeStruct((T, 16), jnp.float32),
        jax.ShapeDtypeStruct((T, 16), jnp.float32),
        jax.ShapeDtypeStruct((1, E), jnp.int32),
        jax.ShapeDtypeStruct((1, 1), jnp.float32),
        jax.ShapeDtypeStruct((1, E), jnp.int32),
    )
    full = lambda shape: pl.BlockSpec(shape, lambda p, i: (0, 0))
    return pl.pallas_call(
        _routing_body,
        grid=(2, NB),
        in_specs=[
            pl.BlockSpec((TB, D), lambda p, i: (i, 0)),
            pl.BlockSpec((D, E), lambda p, i: (0, 0)),
        ],
        out_specs=(
            full((1, T)), full((1, T)), full((1, T)), full((1, T)),
            pl.BlockSpec((T, 16), lambda p, i: (0, 0)),
            pl.BlockSpec((T, 16), lambda p, i: (0, 0)),
            full((1, E)), full((1, 1)), full((1, E)),
        ),
        out_shape=out_shapes,
        scratch_shapes=[
            pltpu.VMEM((1, E), jnp.float32),
            pltpu.VMEM((1, E), jnp.float32),
            pltpu.VMEM((1, E), jnp.float32),
            pltpu.VMEM((1, E), jnp.float32),
            pltpu.VMEM((1, E), jnp.float32),
        ],
        compiler_params=pltpu.CompilerParams(
            dimension_semantics=("arbitrary", "arbitrary")),
    )(xr, wg)


def _ffn_body(fill_ref, a_ref, w1_ref, w2_ref, o_ref, a16_s):
    e = pl.program_id(0)
    k = pl.program_id(1)

    @pl.when(k == 0)
    def _():
        fill = fill_ref[0, e]
        rio = lax.broadcasted_iota(jnp.int32, (C, D), 0)
        a16_s[...] = jnp.where(rio < fill, a_ref[...], 0.0).astype(jnp.bfloat16)

    h = jnp.maximum(
        jnp.dot(a16_s[...], w1_ref[0].astype(jnp.bfloat16),
                preferred_element_type=jnp.float32), 0.0)
    p = jnp.dot(h.astype(jnp.bfloat16), w2_ref[0].astype(jnp.bfloat16),
                preferred_element_type=jnp.float32)

    @pl.when(k == 0)
    def _():
        o_ref[...] = p

    @pl.when(k > 0)
    def _():
        o_ref[...] += p


def _ffn(fill, disp, w1, w2):
    return pl.pallas_call(
        _ffn_body,
        grid=(E, KB),
        in_specs=[
            pl.BlockSpec(memory_space=pltpu.SMEM),
            pl.BlockSpec((C, D), lambda e, k: (e, 0)),
            pl.BlockSpec((1, D, DFFB), lambda e, k: (e, 0, k)),
            pl.BlockSpec((1, DFFB, D), lambda e, k: (e, k, 0)),
        ],
        out_specs=pl.BlockSpec((C, D), lambda e, k: (e, 0)),
        out_shape=jax.ShapeDtypeStruct((EC, D), jnp.float32),
        scratch_shapes=[pltpu.VMEM((C, D), jnp.bfloat16)],
        compiler_params=pltpu.CompilerParams(
            dimension_semantics=("arbitrary", "arbitrary")),
    )(fill, disp, w1, w2)


_NC = 2
_NS = 16
_NW = _NC * _NS

TPW = T // _NW
DCH = 32
CT = 16


def _dispatch(xr, dst1, dst2):
    mesh = plsc.VectorSubcoreMesh(core_axis_name="c", subcore_axis_name="s")

    @functools.partial(
        pl.kernel, mesh=mesh,
        out_type=jax.ShapeDtypeStruct((EC + 8, D), jnp.float32),
        scratch_types=[
            pltpu.VMEM((DCH,), jnp.int32),
            pltpu.VMEM((DCH,), jnp.int32),
            pltpu.VMEM((DCH, D), jnp.float32),
            pltpu.SemaphoreType.DMA,
        ],
    )
    def k(x_hbm, d1_hbm, d2_hbm, out_hbm, i1_v, i2_v, rows_v, sem):
        wid = lax.axis_index("s") * _NC + lax.axis_index("c")
        base = wid * TPW

        def chunk(j, carry):
            off = base + j * DCH
            pltpu.sync_copy(d1_hbm.at[pl.ds(off, DCH)], i1_v)
            pltpu.sync_copy(d2_hbm.at[pl.ds(off, DCH)], i2_v)
            pltpu.sync_copy(x_hbm.at[pl.ds(off, DCH)], rows_v)
            c1 = pltpu.async_copy(rows_v, out_hbm.at[i1_v], sem)
            c2 = pltpu.async_copy(rows_v, out_hbm.at[i2_v], sem)
            c1.wait()
            c2.wait()
            return carry

        lax.fori_loop(0, TPW // DCH, chunk, 0)

    return k(xr, dst1, dst2)


def _combine(eo, slot1, slot2, wc1, wc2):
    mesh = plsc.VectorSubcoreMesh(core_axis_name="c", subcore_axis_name="s")

    @functools.partial(
        pl.kernel, mesh=mesh,
        out_type=jax.ShapeDtypeStruct((T, D), jnp.float32),
        scratch_types=[
            pltpu.VMEM((CT,), jnp.int32),
            pltpu.VMEM((CT,), jnp.int32),
            pltpu.VMEM((CT, 16), jnp.float32),
            pltpu.VMEM((CT, 16), jnp.float32),
            pltpu.VMEM((CT, D), jnp.float32),
            pltpu.VMEM((CT, D), jnp.float32),
            pltpu.VMEM((CT, D), jnp.float32),
            pltpu.SemaphoreType.DMA,
        ],
    )
    def k(eo_hbm, s1_hbm, s2_hbm, w1_hbm, w2_hbm, out_hbm,
          s1_v, s2_v, w1_v, w2_v, r1_v, r2_v, o_v, sem):
        wid = lax.axis_index("s") * _NC + lax.axis_index("c")
        base = wid * TPW

        def chunk(j, carry):
            tb = base + j * CT
            pltpu.sync_copy(s1_hbm.at[pl.ds(tb, CT)], s1_v)
            pltpu.sync_copy(s2_hbm.at[pl.ds(tb, CT)], s2_v)
            pltpu.sync_copy(w1_hbm.at[pl.ds(tb, CT)], w1_v)
            pltpu.sync_copy(w2_hbm.at[pl.ds(tb, CT)], w2_v)
            c1 = pltpu.async_copy(eo_hbm.at[s1_v], r1_v, sem)
            c2 = pltpu.async_copy(eo_hbm.at[s2_v], r2_v, sem)
            c1.wait()
            c2.wait()

            def per_tok(jt):
                wa = w1_v[jt]
                wb = w2_v[jt]
                for kk in range(D // 16):
                    sl = pl.ds(kk * 16, 16)
                    o_v[jt, sl] = r1_v[jt, sl] * wa + r2_v[jt, sl] * wb

            plsc.parallel_loop(0, CT, 1, unroll=2)(per_tok)
            pltpu.sync_copy(o_v, out_hbm.at[pl.ds(tb, CT)])
            return carry

        lax.fori_loop(0, TPW // CT, chunk, 0)

    return k(eo, slot1, slot2, wc1, wc2)


def kernel(hidden_states, wg, w1, w2):
    b, s, d = hidden_states.shape
    xr = hidden_states.reshape(T, D)
    dst1, dst2, slot1, slot2, wc1, wc2, fill, aux, cnt = _routing(xr, wg)
    disp = _dispatch(xr, dst1.reshape(T), dst2.reshape(T))
    eo = _ffn(fill, disp, w1, w2)
    out = _combine(eo, slot1.reshape(T), slot2.reshape(T), wc1, wc2)
    return out.reshape(b, s, d), aux.reshape(()), cnt.reshape(E)


# speedup vs baseline: 1.1151x; 1.1151x over previous
import functools

import jax
import jax.numpy as jnp
from jax import lax
from jax.experimental import pallas as pl
from jax.experimental.pallas import tpu as pltpu
from jax.experimental.pallas import tpu_sc as plsc

E = 8
D = 1024
DFF = 4096
T = 4096
C = 1024
EC = E * C
TRASH = EC
TB = 512
NB = T // TB
KB = 2
DFFB = DFF // KB


def _routing_body(x_ref, wg_ref,
                  dst1_ref, dst2_ref, slot1_ref, slot2_ref, w1_ref, w2_ref,
                  fill_ref, aux_ref, cnt_ref,
                  me_s, c1_s, c2_s, r1_s, r2_s):
    p = pl.program_id(0)
    i = pl.program_id(1)
    x = x_ref[...]
    wg = wg_ref[...]
    logits = jnp.dot(x, wg, preferred_element_type=jnp.float32)
    mx = jnp.max(logits, axis=1, keepdims=True)
    ex = jnp.exp(logits - mx)
    gates = ex / jnp.sum(ex, axis=1, keepdims=True)
    eio = lax.broadcasted_iota(jnp.int32, (TB, E), 1)
    is1 = logits == mx
    idx1 = jnp.min(jnp.where(is1, eio, E), axis=1)
    mask1 = (eio == idx1[:, None]).astype(jnp.float32)
    lneg = jnp.where(mask1 > 0, -1e30, logits)
    mx2 = jnp.max(lneg, axis=1, keepdims=True)
    is2 = lneg == mx2
    idx2 = jnp.min(jnp.where(is2, eio, E), axis=1)
    mask2 = (eio == idx2[:, None]).astype(jnp.float32)

    @pl.when((p == 0) & (i == 0))
    def _():
        me_s[...] = jnp.zeros_like(me_s)
        c1_s[...] = jnp.zeros_like(c1_s)
        c2_s[...] = jnp.zeros_like(c2_s)

    @pl.when(p == 0)
    def _():
        me_s[...] += jnp.sum(gates, axis=0, keepdims=True)
        c1_s[...] += jnp.sum(mask1, axis=0, keepdims=True)
        c2_s[...] += jnp.sum(mask2, axis=0, keepdims=True)

    @pl.when((p == 1) & (i == 0))
    def _():
        r1_s[...] = jnp.zeros_like(r1_s)
        r2_s[...] = c1_s[...]
        aux_ref[...] = jnp.full(
            (1, 1), 8.0 * jnp.sum(me_s[...] * c1_s[...]) / float(T * T),
            jnp.float32)
        cnt_ref[...] = c1_s[...].astype(jnp.int32)
        fill_ref[...] = jnp.minimum(c1_s[...] + c2_s[...],
                                    float(C)).astype(jnp.int32)

    @pl.when(p == 1)
    def _():
        rio = lax.broadcasted_iota(jnp.int32, (TB, TB), 0)
        cio = lax.broadcasted_iota(jnp.int32, (TB, TB), 1)
        ltri = (rio >= cio).astype(jnp.bfloat16)
        inc1 = jnp.dot(ltri, mask1.astype(jnp.bfloat16),
                       preferred_element_type=jnp.float32)
        inc2 = jnp.dot(ltri, mask2.astype(jnp.bfloat16),
                       preferred_element_type=jnp.float32)
        loc1 = inc1 - mask1 + r1_s[...]
        loc2 = inc2 - mask2 + r2_s[...]
        r1_s[...] += jnp.sum(mask1, axis=0, keepdims=True)
        r2_s[...] += jnp.sum(mask2, axis=0, keepdims=True)
        m1d = mask1 * (loc1 < C).astype(jnp.float32)
        m2d = mask2 * (loc2 < C).astype(jnp.float32)
        loc1s = jnp.sum(loc1 * m1d, axis=1)
        loc2s = jnp.sum(loc2 * m2d, axis=1)
        keep1 = jnp.sum(m1d, axis=1)
        keep2 = jnp.sum(m2d, axis=1)
        g1 = jnp.sum(gates * m1d, axis=1)
        g2 = jnp.sum(gates * m2d, axis=1)
        den = jnp.maximum(g1 + g2, jnp.finfo(jnp.float32).eps)
        wc1 = g1 / den * keep1
        wc2 = g2 / den * keep2
        loc1c = jnp.clip(loc1s.astype(jnp.int32), 0, C - 1)
        loc2c = jnp.clip(loc2s.astype(jnp.int32), 0, C - 1)
        slot1 = idx1 * C + loc1c
        slot2 = idx2 * C + loc2c
        sl = pl.ds(i * TB, TB)
        slot1_ref[0, sl] = slot1
        slot2_ref[0, sl] = slot2
        dst1_ref[0, sl] = jnp.where(keep1 > 0, slot1, TRASH)
        dst2_ref[0, sl] = jnp.where(keep2 > 0, slot2, TRASH)
        rsl = pl.ds(i * TB, TB)
        w1_ref[rsl, :] = jnp.broadcast_to(wc1[:, None], (TB, 16))
        w2_ref[rsl, :] = jnp.broadcast_to(wc2[:, None], (TB, 16))


def _routing(xr, wg):
    out_shapes = (
        jax.ShapeDtypeStruct((1, T), jnp.int32),
        jax.ShapeDtypeStruct((1, T), jnp.int32),
        jax.ShapeDtypeStruct((1, T), jnp.int32),
        jax.ShapeDtypeStruct((1, T), jnp.int32),
        jax.ShapeDtypeStruct((T, 16), jnp.float32),
        jax.ShapeDtypeStruct((T, 16), jnp.float32),
        jax.ShapeDtypeStruct((1, E), jnp.int32),
        jax.ShapeDtypeStruct((1, 1), jnp.float32),
        jax.ShapeDtypeStruct((1, E), jnp.int32),
    )
    full = lambda shape: pl.BlockSpec(shape, lambda p, i: (0, 0))
    return pl.pallas_call(
        _routing_body,
        grid=(2, NB),
        in_specs=[
            pl.BlockSpec((TB, D), lambda p, i: (i, 0)),
            pl.BlockSpec((D, E), lambda p, i: (0, 0)),
        ],
        out_specs=(
            full((1, T)), full((1, T)), full((1, T)), full((1, T)),
            pl.BlockSpec((T, 16), lambda p, i: (0, 0)),
            pl.BlockSpec((T, 16), lambda p, i: (0, 0)),
            full((1, E)), full((1, 1)), full((1, E)),
        ),
        out_shape=out_shapes,
        scratch_shapes=[
            pltpu.VMEM((1, E), jnp.float32),
            pltpu.VMEM((1, E), jnp.float32),
            pltpu.VMEM((1, E), jnp.float32),
            pltpu.VMEM((1, E), jnp.float32),
            pltpu.VMEM((1, E), jnp.float32),
        ],
        compiler_params=pltpu.CompilerParams(
            dimension_semantics=("arbitrary", "arbitrary")),
    )(xr, wg)


def _ffn_body(fill_ref, a_ref, w1_ref, w2_ref, o_ref, a16_s):
    e = pl.program_id(0)
    k = pl.program_id(1)

    @pl.when(k == 0)
    def _():
        fill = fill_ref[0, e]
        rio = lax.broadcasted_iota(jnp.int32, (C, D), 0)
        a16_s[...] = jnp.where(rio < fill, a_ref[...], 0.0).astype(jnp.bfloat16)

    h = jnp.maximum(
        jnp.dot(a16_s[...], w1_ref[0].astype(jnp.bfloat16),
                preferred_element_type=jnp.float32), 0.0)
    p = jnp.dot(h.astype(jnp.bfloat16), w2_ref[0].astype(jnp.bfloat16),
                preferred_element_type=jnp.float32)

    @pl.when(k == 0)
    def _():
        o_ref[...] = p

    @pl.when(k > 0)
    def _():
        o_ref[...] += p


def _ffn(fill, disp, w1, w2):
    return pl.pallas_call(
        _ffn_body,
        grid=(E, KB),
        in_specs=[
            pl.BlockSpec(memory_space=pltpu.SMEM),
            pl.BlockSpec((C, D), lambda e, k: (e, 0)),
            pl.BlockSpec((1, D, DFFB), lambda e, k: (e, 0, k)),
            pl.BlockSpec((1, DFFB, D), lambda e, k: (e, k, 0)),
        ],
        out_specs=pl.BlockSpec((C, D), lambda e, k: (e, 0)),
        out_shape=jax.ShapeDtypeStruct((EC, D), jnp.float32),
        scratch_shapes=[pltpu.VMEM((C, D), jnp.bfloat16)],
        compiler_params=pltpu.CompilerParams(
            dimension_semantics=("arbitrary", "arbitrary")),
    )(fill, disp, w1, w2)


_NC = 2
_NS = 16
_NW = _NC * _NS

TPW = T // _NW
DCH = 32
CT = 16


def _dispatch(xr, dst1, dst2):
    mesh = plsc.VectorSubcoreMesh(core_axis_name="c", subcore_axis_name="s")

    @functools.partial(
        pl.kernel, mesh=mesh,
        out_type=jax.ShapeDtypeStruct((EC + 8, D), jnp.float32),
        scratch_types=[
            pltpu.VMEM((DCH,), jnp.int32),
            pltpu.VMEM((DCH,), jnp.int32),
            pltpu.VMEM((DCH, D), jnp.float32),
            pltpu.SemaphoreType.DMA,
        ],
    )
    def k(x_hbm, d1_hbm, d2_hbm, out_hbm, i1_v, i2_v, rows_v, sem):
        wid = lax.axis_index("s") * _NC + lax.axis_index("c")
        base = wid * TPW

        def chunk(j, carry):
            off = base + j * DCH
            pltpu.sync_copy(d1_hbm.at[pl.ds(off, DCH)], i1_v)
            pltpu.sync_copy(d2_hbm.at[pl.ds(off, DCH)], i2_v)
            pltpu.sync_copy(x_hbm.at[pl.ds(off, DCH)], rows_v)
            c1 = pltpu.async_copy(rows_v, out_hbm.at[i1_v], sem)
            c2 = pltpu.async_copy(rows_v, out_hbm.at[i2_v], sem)
            c1.wait()
            c2.wait()
            return carry

        lax.fori_loop(0, TPW // DCH, chunk, 0)

    return k(xr, dst1, dst2)


def _combine(eo, slot1, slot2, wc1, wc2):
    mesh = plsc.VectorSubcoreMesh(core_axis_name="c", subcore_axis_name="s")

    @functools.partial(
        pl.kernel, mesh=mesh,
        out_type=jax.ShapeDtypeStruct((T, D), jnp.float32),
        scratch_types=[
            pltpu.VMEM((CT,), jnp.int32),
            pltpu.VMEM((CT,), jnp.int32),
            pltpu.VMEM((CT, 16), jnp.float32),
            pltpu.VMEM((CT, 16), jnp.float32),
            pltpu.VMEM((CT, D), jnp.float32),
            pltpu.VMEM((CT, D), jnp.float32),
            pltpu.VMEM((CT, D), jnp.float32),
            pltpu.SemaphoreType.DMA,
        ],
    )
    def k(eo_hbm, s1_hbm, s2_hbm, w1_hbm, w2_hbm, out_hbm,
          s1_v, s2_v, w1_v, w2_v, r1_v, r2_v, o_v, sem):
        wid = lax.axis_index("s") * _NC + lax.axis_index("c")
        base = wid * TPW

        def chunk(j, carry):
            tb = base + j * CT
            pltpu.sync_copy(s1_hbm.at[pl.ds(tb, CT)], s1_v)
            pltpu.sync_copy(s2_hbm.at[pl.ds(tb, CT)], s2_v)
            pltpu.sync_copy(w1_hbm.at[pl.ds(tb, CT)], w1_v)
            pltpu.sync_copy(w2_hbm.at[pl.ds(tb, CT)], w2_v)
            c1 = pltpu.async_copy(eo_hbm.at[s1_v], r1_v, sem)
            c2 = pltpu.async_copy(eo_hbm.at[s2_v], r2_v, sem)
            c1.wait()
            c2.wait()

            def per_tok(jt, c_):
                wa = w1_v[jt]
                wb = w2_v[jt]
                for kk in range(D // 16):
                    sl = pl.ds(kk * 16, 16)
                    o_v[jt, sl] = r1_v[jt, sl] * wa + r2_v[jt, sl] * wb
                return c_

            lax.fori_loop(0, CT, per_tok, 0)
            pltpu.sync_copy(o_v, out_hbm.at[pl.ds(tb, CT)])
            return carry

        lax.fori_loop(0, TPW // CT, chunk, 0)

    return k(eo, slot1, slot2, wc1, wc2)


def kernel(hidden_states, wg, w1, w2):
    b, s, d = hidden_states.shape
    xr = hidden_states.reshape(T, D)
    dst1, dst2, slot1, slot2, wc1, wc2, fill, aux, cnt = _routing(xr, wg)
    disp = _dispatch(xr, dst1.reshape(T), dst2.reshape(T))
    eo = _ffn(fill, disp, w1, w2)
    out = _combine(eo, slot1.reshape(T), slot2.reshape(T), wc1, wc2)
    return out.reshape(b, s, d), aux.reshape(()), cnt.reshape(E)


# speedup vs baseline: 1.1485x; 1.0300x over previous
import functools

import jax
import jax.numpy as jnp
from jax import lax
from jax.experimental import pallas as pl
from jax.experimental.pallas import tpu as pltpu
from jax.experimental.pallas import tpu_sc as plsc

E = 8
D = 1024
DFF = 4096
T = 4096
C = 1024
EC = E * C
TRASH = EC
TB = 512
NB = T // TB
KB = 2
DFFB = DFF // KB


def _routing_body(x_ref, wg_ref,
                  dst1_ref, dst2_ref, slot1_ref, slot2_ref, w1_ref, w2_ref,
                  fill_ref, aux_ref, cnt_ref,
                  me_s, c1_s, c2_s, r1_s, r2_s):
    p = pl.program_id(0)
    i = pl.program_id(1)
    x = x_ref[...]
    wg = wg_ref[...]
    logits = jnp.dot(x, wg, preferred_element_type=jnp.float32)
    mx = jnp.max(logits, axis=1, keepdims=True)
    ex = jnp.exp(logits - mx)
    gates = ex / jnp.sum(ex, axis=1, keepdims=True)
    eio = lax.broadcasted_iota(jnp.int32, (TB, E), 1)
    is1 = logits == mx
    idx1 = jnp.min(jnp.where(is1, eio, E), axis=1)
    mask1 = (eio == idx1[:, None]).astype(jnp.float32)
    lneg = jnp.where(mask1 > 0, -1e30, logits)
    mx2 = jnp.max(lneg, axis=1, keepdims=True)
    is2 = lneg == mx2
    idx2 = jnp.min(jnp.where(is2, eio, E), axis=1)
    mask2 = (eio == idx2[:, None]).astype(jnp.float32)

    @pl.when((p == 0) & (i == 0))
    def _():
        me_s[...] = jnp.zeros_like(me_s)
        c1_s[...] = jnp.zeros_like(c1_s)
        c2_s[...] = jnp.zeros_like(c2_s)

    @pl.when(p == 0)
    def _():
        me_s[...] += jnp.sum(gates, axis=0, keepdims=True)
        c1_s[...] += jnp.sum(mask1, axis=0, keepdims=True)
        c2_s[...] += jnp.sum(mask2, axis=0, keepdims=True)

    @pl.when((p == 1) & (i == 0))
    def _():
        r1_s[...] = jnp.zeros_like(r1_s)
        r2_s[...] = c1_s[...]
        aux_ref[...] = jnp.full(
            (1, 1), 8.0 * jnp.sum(me_s[...] * c1_s[...]) / float(T * T),
            jnp.float32)
        cnt_ref[...] = c1_s[...].astype(jnp.int32)
        fill_ref[...] = jnp.minimum(c1_s[...] + c2_s[...],
                                    float(C)).astype(jnp.int32)

    @pl.when(p == 1)
    def _():
        rio = lax.broadcasted_iota(jnp.int32, (TB, TB), 0)
        cio = lax.broadcasted_iota(jnp.int32, (TB, TB), 1)
        ltri = (rio >= cio).astype(jnp.bfloat16)
        inc1 = jnp.dot(ltri, mask1.astype(jnp.bfloat16),
                       preferred_element_type=jnp.float32)
        inc2 = jnp.dot(ltri, mask2.astype(jnp.bfloat16),
                       preferred_element_type=jnp.float32)
        loc1 = inc1 - mask1 + r1_s[...]
        loc2 = inc2 - mask2 + r2_s[...]
        r1_s[...] += jnp.sum(mask1, axis=0, keepdims=True)
        r2_s[...] += jnp.sum(mask2, axis=0, keepdims=True)
        m1d = mask1 * (loc1 < C).astype(jnp.float32)
        m2d = mask2 * (loc2 < C).astype(jnp.float32)
        loc1s = jnp.sum(loc1 * m1d, axis=1)
        loc2s = jnp.sum(loc2 * m2d, axis=1)
        keep1 = jnp.sum(m1d, axis=1)
        keep2 = jnp.sum(m2d, axis=1)
        g1 = jnp.sum(gates * m1d, axis=1)
        g2 = jnp.sum(gates * m2d, axis=1)
        den = jnp.maximum(g1 + g2, jnp.finfo(jnp.float32).eps)
        wc1 = g1 / den * keep1
        wc2 = g2 / den * keep2
        loc1c = jnp.clip(loc1s.astype(jnp.int32), 0, C - 1)
        loc2c = jnp.clip(loc2s.astype(jnp.int32), 0, C - 1)
        slot1 = idx1 * C + loc1c
        slot2 = idx2 * C + loc2c
        sl = pl.ds(i * TB, TB)
        slot1_ref[0, sl] = slot1
        slot2_ref[0, sl] = slot2
        dst1_ref[0, sl] = jnp.where(keep1 > 0, slot1, TRASH)
        dst2_ref[0, sl] = jnp.where(keep2 > 0, slot2, TRASH)
        rsl = pl.ds(i * TB, TB)
        w1_ref[rsl, :] = jnp.broadcast_to(wc1[:, None], (TB, 16))
        w2_ref[rsl, :] = jnp.broadcast_to(wc2[:, None], (TB, 16))


def _routing(xr, wg):
    out_shapes = (
        jax.ShapeDtypeStruct((1, T), jnp.int32),
        jax.ShapeDtypeStruct((1, T), jnp.int32),
        jax.ShapeDtypeStruct((1, T), jnp.int32),
        jax.ShapeDtypeStruct((1, T), jnp.int32),
        jax.ShapeDtypeStruct((T, 16), jnp.float32),
        jax.ShapeDtypeStruct((T, 16), jnp.float32),
        jax.ShapeDtypeStruct((1, E), jnp.int32),
        jax.ShapeDtypeStruct((1, 1), jnp.float32),
        jax.ShapeDtypeStruct((1, E), jnp.int32),
    )
    full = lambda shape: pl.BlockSpec(shape, lambda p, i: (0, 0))
    return pl.pallas_call(
        _routing_body,
        grid=(2, NB),
        in_specs=[
            pl.BlockSpec((TB, D), lambda p, i: (i, 0)),
            pl.BlockSpec((D, E), lambda p, i: (0, 0)),
        ],
        out_specs=(
            full((1, T)), full((1, T)), full((1, T)), full((1, T)),
            pl.BlockSpec((T, 16), lambda p, i: (0, 0)),
            pl.BlockSpec((T, 16), lambda p, i: (0, 0)),
            full((1, E)), full((1, 1)), full((1, E)),
        ),
        out_shape=out_shapes,
        scratch_shapes=[
            pltpu.VMEM((1, E), jnp.float32),
            pltpu.VMEM((1, E), jnp.float32),
            pltpu.VMEM((1, E), jnp.float32),
            pltpu.VMEM((1, E), jnp.float32),
            pltpu.VMEM((1, E), jnp.float32),
        ],
        compiler_params=pltpu.CompilerParams(
            dimension_semantics=("arbitrary", "arbitrary")),
    )(xr, wg)


def _ffn_body(fill_ref, a_ref, w1_ref, w2_ref, o_ref, a16_s):
    e = pl.program_id(0)
    k = pl.program_id(1)

    @pl.when(k == 0)
    def _():
        fill = fill_ref[0, e]
        rio = lax.broadcasted_iota(jnp.int32, (C, D), 0)
        a16_s[...] = jnp.where(rio < fill, a_ref[...], 0.0).astype(jnp.bfloat16)

    h = jnp.maximum(
        jnp.dot(a16_s[...], w1_ref[0].astype(jnp.bfloat16),
                preferred_element_type=jnp.float32), 0.0)
    p = jnp.dot(h.astype(jnp.bfloat16), w2_ref[0].astype(jnp.bfloat16),
                preferred_element_type=jnp.float32)

    @pl.when(k == 0)
    def _():
        o_ref[...] = p

    @pl.when(k > 0)
    def _():
        o_ref[...] += p


def _ffn(fill, disp, w1, w2):
    return pl.pallas_call(
        _ffn_body,
        grid=(E, KB),
        in_specs=[
            pl.BlockSpec(memory_space=pltpu.SMEM),
            pl.BlockSpec((C, D), lambda e, k: (e, 0)),
            pl.BlockSpec((1, D, DFFB), lambda e, k: (e, 0, k)),
            pl.BlockSpec((1, DFFB, D), lambda e, k: (e, k, 0)),
        ],
        out_specs=pl.BlockSpec((C, D), lambda e, k: (e, 0)),
        out_shape=jax.ShapeDtypeStruct((EC, D), jnp.float32),
        scratch_shapes=[pltpu.VMEM((C, D), jnp.bfloat16)],
        compiler_params=pltpu.CompilerParams(
            dimension_semantics=("arbitrary", "arbitrary")),
    )(fill, disp, w1, w2)


_NC = 2
_NS = 16
_NW = _NC * _NS

TPW = T // _NW
DCH = 32
CT = 32


def _dispatch(xr, dst1, dst2):
    mesh = plsc.VectorSubcoreMesh(core_axis_name="c", subcore_axis_name="s")

    @functools.partial(
        pl.kernel, mesh=mesh,
        out_type=jax.ShapeDtypeStruct((EC + 8, D), jnp.float32),
        scratch_types=[
            pltpu.VMEM((DCH,), jnp.int32),
            pltpu.VMEM((DCH,), jnp.int32),
            pltpu.VMEM((DCH, D), jnp.float32),
            pltpu.SemaphoreType.DMA,
        ],
    )
    def k(x_hbm, d1_hbm, d2_hbm, out_hbm, i1_v, i2_v, rows_v, sem):
        wid = lax.axis_index("s") * _NC + lax.axis_index("c")
        base = wid * TPW

        def chunk(j, carry):
            off = base + j * DCH
            pltpu.sync_copy(d1_hbm.at[pl.ds(off, DCH)], i1_v)
            pltpu.sync_copy(d2_hbm.at[pl.ds(off, DCH)], i2_v)
            pltpu.sync_copy(x_hbm.at[pl.ds(off, DCH)], rows_v)
            c1 = pltpu.async_copy(rows_v, out_hbm.at[i1_v], sem)
            c2 = pltpu.async_copy(rows_v, out_hbm.at[i2_v], sem)
            c1.wait()
            c2.wait()
            return carry

        lax.fori_loop(0, TPW // DCH, chunk, 0)

    return k(xr, dst1, dst2)


def _combine(eo, slot1, slot2, wc1, wc2):
    mesh = plsc.VectorSubcoreMesh(core_axis_name="c", subcore_axis_name="s")

    @functools.partial(
        pl.kernel, mesh=mesh,
        out_type=jax.ShapeDtypeStruct((T, D), jnp.float32),
        scratch_types=[
            pltpu.VMEM((CT,), jnp.int32),
            pltpu.VMEM((CT,), jnp.int32),
            pltpu.VMEM((CT, 16), jnp.float32),
            pltpu.VMEM((CT, 16), jnp.float32),
            pltpu.VMEM((CT, D), jnp.float32),
            pltpu.VMEM((CT, D), jnp.float32),
            pltpu.VMEM((CT, D), jnp.float32),
            pltpu.SemaphoreType.DMA,
        ],
    )
    def k(eo_hbm, s1_hbm, s2_hbm, w1_hbm, w2_hbm, out_hbm,
          s1_v, s2_v, w1_v, w2_v, r1_v, r2_v, o_v, sem):
        wid = lax.axis_index("s") * _NC + lax.axis_index("c")
        base = wid * TPW

        def chunk(j, carry):
            tb = base + j * CT
            pltpu.sync_copy(s1_hbm.at[pl.ds(tb, CT)], s1_v)
            pltpu.sync_copy(s2_hbm.at[pl.ds(tb, CT)], s2_v)
            pltpu.sync_copy(w1_hbm.at[pl.ds(tb, CT)], w1_v)
            pltpu.sync_copy(w2_hbm.at[pl.ds(tb, CT)], w2_v)
            c1 = pltpu.async_copy(eo_hbm.at[s1_v], r1_v, sem)
            c2 = pltpu.async_copy(eo_hbm.at[s2_v], r2_v, sem)
            c1.wait()
            c2.wait()

            def per_tok(jt, c_):
                wa = w1_v[jt]
                wb = w2_v[jt]
                for kk in range(D // 16):
                    sl = pl.ds(kk * 16, 16)
                    o_v[jt, sl] = r1_v[jt, sl] * wa + r2_v[jt, sl] * wb
                return c_

            lax.fori_loop(0, CT, per_tok, 0)
            pltpu.sync_copy(o_v, out_hbm.at[pl.ds(tb, CT)])
            return carry

        lax.fori_loop(0, TPW // CT, chunk, 0)

    return k(eo, slot1, slot2, wc1, wc2)


def kernel(hidden_states, wg, w1, w2):
    b, s, d = hidden_states.shape
    xr = hidden_states.reshape(T, D)
    dst1, dst2, slot1, slot2, wc1, wc2, fill, aux, cnt = _routing(xr, wg)
    disp = _dispatch(xr, dst1.reshape(T), dst2.reshape(T))
    eo = _ffn(fill, disp, w1, w2)
    out = _combine(eo, slot1.reshape(T), slot2.reshape(T), wc1, wc2)
    return out.reshape(b, s, d), aux.reshape(()), cnt.reshape(E)


# speedup vs baseline: 1.1543x; 1.0050x over previous
import functools

import jax
import jax.numpy as jnp
from jax import lax
from jax.experimental import pallas as pl
from jax.experimental.pallas import tpu as pltpu
from jax.experimental.pallas import tpu_sc as plsc

E = 8
D = 1024
DFF = 4096
T = 4096
C = 1024
EC = E * C
TRASH = EC
TB = 512
NB = T // TB
KB = 2
DFFB = DFF // KB


def _routing_body(x_ref, wg_ref,
                  dst1_ref, dst2_ref, slot1_ref, slot2_ref, w1_ref, w2_ref,
                  fill_ref, aux_ref, cnt_ref,
                  me_s, c1_s, c2_s, r1_s, r2_s):
    p = pl.program_id(0)
    i = pl.program_id(1)
    x = x_ref[...]
    wg = wg_ref[...]
    logits = jnp.dot(x, wg, preferred_element_type=jnp.float32)
    mx = jnp.max(logits, axis=1, keepdims=True)
    ex = jnp.exp(logits - mx)
    gates = ex / jnp.sum(ex, axis=1, keepdims=True)
    eio = lax.broadcasted_iota(jnp.int32, (TB, E), 1)
    is1 = logits == mx
    idx1 = jnp.min(jnp.where(is1, eio, E), axis=1)
    mask1 = (eio == idx1[:, None]).astype(jnp.float32)
    lneg = jnp.where(mask1 > 0, -1e30, logits)
    mx2 = jnp.max(lneg, axis=1, keepdims=True)
    is2 = lneg == mx2
    idx2 = jnp.min(jnp.where(is2, eio, E), axis=1)
    mask2 = (eio == idx2[:, None]).astype(jnp.float32)

    @pl.when((p == 0) & (i == 0))
    def _():
        me_s[...] = jnp.zeros_like(me_s)
        c1_s[...] = jnp.zeros_like(c1_s)
        c2_s[...] = jnp.zeros_like(c2_s)

    @pl.when(p == 0)
    def _():
        me_s[...] += jnp.sum(gates, axis=0, keepdims=True)
        c1_s[...] += jnp.sum(mask1, axis=0, keepdims=True)
        c2_s[...] += jnp.sum(mask2, axis=0, keepdims=True)

    @pl.when((p == 1) & (i == 0))
    def _():
        r1_s[...] = jnp.zeros_like(r1_s)
        r2_s[...] = c1_s[...]
        aux_ref[...] = jnp.full(
            (1, 1), 8.0 * jnp.sum(me_s[...] * c1_s[...]) / float(T * T),
            jnp.float32)
        cnt_ref[...] = c1_s[...].astype(jnp.int32)
        fill_ref[...] = jnp.minimum(c1_s[...] + c2_s[...],
                                    float(C)).astype(jnp.int32)

    @pl.when(p == 1)
    def _():
        rio = lax.broadcasted_iota(jnp.int32, (TB, TB), 0)
        cio = lax.broadcasted_iota(jnp.int32, (TB, TB), 1)
        ltri = (rio >= cio).astype(jnp.bfloat16)
        inc1 = jnp.dot(ltri, mask1.astype(jnp.bfloat16),
                       preferred_element_type=jnp.float32)
        inc2 = jnp.dot(ltri, mask2.astype(jnp.bfloat16),
                       preferred_element_type=jnp.float32)
        loc1 = inc1 - mask1 + r1_s[...]
        loc2 = inc2 - mask2 + r2_s[...]
        r1_s[...] += jnp.sum(mask1, axis=0, keepdims=True)
        r2_s[...] += jnp.sum(mask2, axis=0, keepdims=True)
        m1d = mask1 * (loc1 < C).astype(jnp.float32)
        m2d = mask2 * (loc2 < C).astype(jnp.float32)
        loc1s = jnp.sum(loc1 * m1d, axis=1)
        loc2s = jnp.sum(loc2 * m2d, axis=1)
        keep1 = jnp.sum(m1d, axis=1)
        keep2 = jnp.sum(m2d, axis=1)
        g1 = jnp.sum(gates * m1d, axis=1)
        g2 = jnp.sum(gates * m2d, axis=1)
        den = jnp.maximum(g1 + g2, jnp.finfo(jnp.float32).eps)
        wc1 = g1 / den * keep1
        wc2 = g2 / den * keep2
        loc1c = jnp.clip(loc1s.astype(jnp.int32), 0, C - 1)
        loc2c = jnp.clip(loc2s.astype(jnp.int32), 0, C - 1)
        slot1 = idx1 * C + loc1c
        slot2 = idx2 * C + loc2c
        sl = pl.ds(i * TB, TB)
        slot1_ref[0, sl] = slot1
        slot2_ref[0, sl] = slot2
        dst1_ref[0, sl] = jnp.where(keep1 > 0, slot1, TRASH)
        dst2_ref[0, sl] = jnp.where(keep2 > 0, slot2, TRASH)
        rsl = pl.ds(i * TB, TB)
        w1_ref[rsl, :] = jnp.broadcast_to(wc1[:, None], (TB, 16))
        w2_ref[rsl, :] = jnp.broadcast_to(wc2[:, None], (TB, 16))


def _routing(xr, wg):
    out_shapes = (
        jax.ShapeDtypeStruct((1, T), jnp.int32),
        jax.ShapeDtypeStruct((1, T), jnp.int32),
        jax.ShapeDtypeStruct((1, T), jnp.int32),
        jax.ShapeDtypeStruct((1, T), jnp.int32),
        jax.ShapeDtypeStruct((T, 16), jnp.float32),
        jax.ShapeDtypeStruct((T, 16), jnp.float32),
        jax.ShapeDtypeStruct((1, E), jnp.int32),
        jax.ShapeDtypeStruct((1, 1), jnp.float32),
        jax.ShapeDtypeStruct((1, E), jnp.int32),
    )
    full = lambda shape: pl.BlockSpec(shape, lambda p, i: (0, 0))
    return pl.pallas_call(
        _routing_body,
        grid=(2, NB),
        in_specs=[
            pl.BlockSpec((TB, D), lambda p, i: (i, 0)),
            pl.BlockSpec((D, E), lambda p, i: (0, 0)),
        ],
        out_specs=(
            full((1, T)), full((1, T)), full((1, T)), full((1, T)),
            pl.BlockSpec((T, 16), lambda p, i: (0, 0)),
            pl.BlockSpec((T, 16), lambda p, i: (0, 0)),
            full((1, E)), full((1, 1)), full((1, E)),
        ),
        out_shape=out_shapes,
        scratch_shapes=[
            pltpu.VMEM((1, E), jnp.float32),
            pltpu.VMEM((1, E), jnp.float32),
            pltpu.VMEM((1, E), jnp.float32),
            pltpu.VMEM((1, E), jnp.float32),
            pltpu.VMEM((1, E), jnp.float32),
        ],
        compiler_params=pltpu.CompilerParams(
            dimension_semantics=("arbitrary", "arbitrary")),
    )(xr, wg)


def _ffn_body(fill_ref, a_ref, w1_ref, w2_ref, o_ref, a16_s):
    e = pl.program_id(0)
    k = pl.program_id(1)

    @pl.when(k == 0)
    def _():
        fill = fill_ref[0, e]
        rio = lax.broadcasted_iota(jnp.int32, (C, D), 0)
        a16_s[...] = jnp.where(rio < fill, a_ref[...], 0.0).astype(jnp.bfloat16)

    h = jnp.maximum(
        jnp.dot(a16_s[...], w1_ref[0].astype(jnp.bfloat16),
                preferred_element_type=jnp.float32), 0.0)
    p = jnp.dot(h.astype(jnp.bfloat16), w2_ref[0].astype(jnp.bfloat16),
                preferred_element_type=jnp.float32)

    @pl.when(k == 0)
    def _():
        o_ref[...] = p

    @pl.when(k > 0)
    def _():
        o_ref[...] += p


def _ffn(fill, disp, w1, w2):
    return pl.pallas_call(
        _ffn_body,
        grid=(E, KB),
        in_specs=[
            pl.BlockSpec(memory_space=pltpu.SMEM),
            pl.BlockSpec((C, D), lambda e, k: (e, 0)),
            pl.BlockSpec((1, D, DFFB), lambda e, k: (e, 0, k)),
            pl.BlockSpec((1, DFFB, D), lambda e, k: (e, k, 0)),
        ],
        out_specs=pl.BlockSpec((C, D), lambda e, k: (e, 0)),
        out_shape=jax.ShapeDtypeStruct((EC, D), jnp.float32),
        scratch_shapes=[pltpu.VMEM((C, D), jnp.bfloat16)],
        compiler_params=pltpu.CompilerParams(
            dimension_semantics=("arbitrary", "arbitrary")),
    )(fill, disp, w1, w2)


_NC = 2
_NS = 16
_NW = _NC * _NS

TPW = T // _NW
DCH = 64
CT = 32


def _dispatch(xr, dst1, dst2):
    mesh = plsc.VectorSubcoreMesh(core_axis_name="c", subcore_axis_name="s")

    @functools.partial(
        pl.kernel, mesh=mesh,
        out_type=jax.ShapeDtypeStruct((EC + 8, D), jnp.float32),
        scratch_types=[
            pltpu.VMEM((DCH,), jnp.int32),
            pltpu.VMEM((DCH,), jnp.int32),
            pltpu.VMEM((DCH, D), jnp.float32),
            pltpu.SemaphoreType.DMA,
        ],
    )
    def k(x_hbm, d1_hbm, d2_hbm, out_hbm, i1_v, i2_v, rows_v, sem):
        wid = lax.axis_index("s") * _NC + lax.axis_index("c")
        base = wid * TPW

        def chunk(j, carry):
            off = base + j * DCH
            pltpu.sync_copy(d1_hbm.at[pl.ds(off, DCH)], i1_v)
            pltpu.sync_copy(d2_hbm.at[pl.ds(off, DCH)], i2_v)
            pltpu.sync_copy(x_hbm.at[pl.ds(off, DCH)], rows_v)
            c1 = pltpu.async_copy(rows_v, out_hbm.at[i1_v], sem)
            c2 = pltpu.async_copy(rows_v, out_hbm.at[i2_v], sem)
            c1.wait()
            c2.wait()
            return carry

        lax.fori_loop(0, TPW // DCH, chunk, 0)

    return k(xr, dst1, dst2)


def _combine(eo, slot1, slot2, wc1, wc2):
    mesh = plsc.VectorSubcoreMesh(core_axis_name="c", subcore_axis_name="s")

    @functools.partial(
        pl.kernel, mesh=mesh,
        out_type=jax.ShapeDtypeStruct((T, D), jnp.float32),
        scratch_types=[
            pltpu.VMEM((CT,), jnp.int32),
            pltpu.VMEM((CT,), jnp.int32),
            pltpu.VMEM((CT, 16), jnp.float32),
            pltpu.VMEM((CT, 16), jnp.float32),
            pltpu.VMEM((CT, D), jnp.float32),
            pltpu.VMEM((CT, D), jnp.float32),
            pltpu.SemaphoreType.DMA,
        ],
    )
    def k(eo_hbm, s1_hbm, s2_hbm, w1_hbm, w2_hbm, out_hbm,
          s1_v, s2_v, w1_v, w2_v, r1_v, r2_v, sem):
        wid = lax.axis_index("s") * _NC + lax.axis_index("c")
        base = wid * TPW

        def chunk(j, carry):
            tb = base + j * CT
            pltpu.sync_copy(s1_hbm.at[pl.ds(tb, CT)], s1_v)
            pltpu.sync_copy(s2_hbm.at[pl.ds(tb, CT)], s2_v)
            pltpu.sync_copy(w1_hbm.at[pl.ds(tb, CT)], w1_v)
            pltpu.sync_copy(w2_hbm.at[pl.ds(tb, CT)], w2_v)
            c1 = pltpu.async_copy(eo_hbm.at[s1_v], r1_v, sem)
            c2 = pltpu.async_copy(eo_hbm.at[s2_v], r2_v, sem)
            c1.wait()
            c2.wait()

            def per_tok(jt, c_):
                wa = w1_v[jt]
                wb = w2_v[jt]
                for kk in range(D // 16):
                    sl = pl.ds(kk * 16, 16)
                    r1_v[jt, sl] = r1_v[jt, sl] * wa + r2_v[jt, sl] * wb
                return c_

            lax.fori_loop(0, CT, per_tok, 0)
            pltpu.sync_copy(r1_v, out_hbm.at[pl.ds(tb, CT)])
            return carry

        lax.fori_loop(0, TPW // CT, chunk, 0)

    return k(eo, slot1, slot2, wc1, wc2)


def kernel(hidden_states, wg, w1, w2):
    b, s, d = hidden_states.shape
    xr = hidden_states.reshape(T, D)
    dst1, dst2, slot1, slot2, wc1, wc2, fill, aux, cnt = _routing(xr, wg)
    disp = _dispatch(xr, dst1.reshape(T), dst2.reshape(T))
    eo = _ffn(fill, disp, w1, w2)
    out = _combine(eo, slot1.reshape(T), slot2.reshape(T), wc1, wc2)
    return out.reshape(b, s, d), aux.reshape(()), cnt.reshape(E)
